# baseline (speedup 1.0000x reference)
import functools

import jax
import jax.numpy as jnp
from jax import lax
from jax.experimental import pallas as pl
from jax.experimental.pallas import tpu as pltpu
from jax.experimental.pallas import tpu_sc as plsc

D = 32
NC, NS = 2, 16
NW = NC * NS
CHUNK = 128

mesh = plsc.VectorSubcoreMesh(core_axis_name="c", subcore_axis_name="s")


def _make_emb(B):
    b_per_w = B // NW
    nch = b_per_w // CHUNK

    @functools.partial(
        pl.kernel,
        mesh=mesh,
        out_type=jax.ShapeDtypeStruct((B, D), jnp.float32),
        scratch_types=[
            pltpu.VMEM((nch, CHUNK), jnp.int32),
            pltpu.VMEM((CHUNK, D), jnp.float32),
            pltpu.SemaphoreType.DMA,
        ],
        compiler_params=pltpu.CompilerParams(use_tc_tiling_on_sc=False),
    )
    def _emb(idx_hbm, table_hbm, out_hbm, idx_v, rows_v, sem):
        wid = lax.axis_index("s") * NC + lax.axis_index("c")
        base = wid * b_per_w
        pltpu.sync_copy(idx_hbm.at[wid], idx_v)

        def body(j, carry):
            pltpu.async_copy(table_hbm.at[idx_v.at[j]], rows_v, sem).wait()
            pltpu.sync_copy(rows_v, out_hbm.at[pl.ds(base + j * CHUNK, CHUNK)])
            return carry

        lax.fori_loop(0, nch, body, 0)

    return _emb


def kernel(indexes, table):
    B = indexes.size
    idx = indexes.reshape(NW, B // NW // CHUNK, CHUNK).astype(jnp.int32)
    out = _make_emb(B)(idx, table)
    return out.reshape(*indexes.shape, D)


# speedup vs baseline: 1.0449x; 1.0449x over previous
import functools

import jax
import jax.numpy as jnp
from jax import lax
from jax.experimental import pallas as pl
from jax.experimental.pallas import tpu as pltpu
from jax.experimental.pallas import tpu_sc as plsc

D = 32
NC, NS = 2, 16
NW = NC * NS
CHUNK = 128
NBUF = 5

mesh = plsc.VectorSubcoreMesh(core_axis_name="c", subcore_axis_name="s")


def _make_emb(B):
    b_per_w = B // NW
    nch = b_per_w // CHUNK
    assert nch % NBUF == 0

    @functools.partial(
        pl.kernel,
        mesh=mesh,
        out_type=jax.ShapeDtypeStruct((B, D), jnp.float32),
        scratch_types=[
            pltpu.VMEM((nch, CHUNK), jnp.int32),
            pltpu.VMEM((NBUF, CHUNK, D), jnp.float32),
            [pltpu.SemaphoreType.DMA] * NBUF,
        ],
        compiler_params=pltpu.CompilerParams(use_tc_tiling_on_sc=False),
    )
    def _emb(idx_hbm, table_hbm, out_hbm, idx_v, rows_v, sems):
        wid = lax.axis_index("s") * NC + lax.axis_index("c")
        base = wid * b_per_w
        pltpu.sync_copy(idx_hbm.at[wid], idx_v)

        for b in range(NBUF):
            pltpu.async_copy(table_hbm.at[idx_v.at[b]], rows_v.at[b], sems[b])

        @pl.loop(0, nch, step=NBUF)
        def _(g):
            for b in range(NBUF):
                j = g + b
                pltpu.make_async_copy(
                    table_hbm.at[idx_v.at[0]], rows_v.at[b], sems[b]
                ).wait()
                pltpu.sync_copy(
                    rows_v.at[b], out_hbm.at[pl.ds(base + j * CHUNK, CHUNK)]
                )
                nxt = j + NBUF

                @pl.when(nxt < nch)
                def _():
                    pltpu.async_copy(
                        table_hbm.at[idx_v.at[nxt]], rows_v.at[b], sems[b]
                    )

    return _emb


def kernel(indexes, table):
    B = indexes.size
    idx = indexes.reshape(NW, B // NW // CHUNK, CHUNK).astype(jnp.int32)
    out = _make_emb(B)(idx, table)
    return out.reshape(*indexes.shape, D)


# speedup vs baseline: 1.2125x; 1.1603x over previous
import functools

import jax
import jax.numpy as jnp
from jax import lax
from jax.experimental import pallas as pl
from jax.experimental.pallas import tpu as pltpu
from jax.experimental.pallas import tpu_sc as plsc

D = 32
NC, NS = 2, 16
NW = NC * NS
BB = 4096 // NW
S = 50
RPE = 128 // D

mesh = plsc.VectorSubcoreMesh(core_axis_name="c", subcore_axis_name="s")


@functools.partial(
    pl.kernel,
    mesh=mesh,
    out_type=jax.ShapeDtypeStruct((S, D, 4096), jnp.float32),
    scratch_types=[
        pltpu.VMEM((S, BB), jnp.int32),
        pltpu.VMEM((S, BB), jnp.int32),
        pltpu.VMEM((S, BB), jnp.int32),
        pltpu.VMEM((2, BB, 128), jnp.float32),
        pltpu.VMEM((2, D, BB), jnp.float32),
        pltpu.SemaphoreType.DMA,
        pltpu.SemaphoreType.DMA,
        pltpu.SemaphoreType.DMA,
        pltpu.SemaphoreType.DMA,
    ],
    compiler_params=pltpu.CompilerParams(needs_layout_passes=False),
)
def _emb(idx_hbm, tbl_hbm, out_hbm, idx_v, row_v, sub_v, gbuf, obuf,
         gsem0, gsem1, osem0, osem1):
    wid = lax.axis_index("s") * NC + lax.axis_index("c")
    base = wid * BB
    pltpu.sync_copy(idx_hbm.at[:, pl.ds(base, BB)], idx_v)

    @pl.loop(0, S)
    def _(s):
        for g in range(BB // 16):
            v = idx_v[s, pl.ds(g * 16, 16)]
            row_v[s, pl.ds(g * 16, 16)] = lax.shift_right_logical(v, 2)
            sub_v[s, pl.ds(g * 16, 16)] = lax.shift_left(
                lax.bitwise_and(v, 3), 5)

    gsems = [gsem0, gsem1]
    osems = [osem0, osem1]

    def start_gather(s, buf):
        pltpu.async_copy(tbl_hbm.at[row_v.at[s]], gbuf.at[buf], gsems[buf])

    def wait_gather(buf):
        pltpu.make_async_copy(
            tbl_hbm.at[row_v.at[0]], gbuf.at[buf], gsems[buf]).wait()

    def start_out(s, buf):
        pltpu.async_copy(
            obuf.at[buf], out_hbm.at[s, :, pl.ds(base, BB)], osems[buf])

    def wait_out(buf):
        pltpu.make_async_copy(
            obuf.at[buf], out_hbm.at[0, :, pl.ds(base, BB)],
            osems[buf]).wait()

    start_gather(0, 0)
    start_gather(1, 1)

    iota = lax.iota(jnp.int32, 16)

    @pl.loop(0, S, step=2)
    def _(s0):
        for buf in range(2):
            s = s0 + buf
            wait_gather(buf)

            @pl.when(s >= 2)
            def _():
                wait_out(buf)

            for g in range(BB // 16):
                rows = g * 16 + iota
                cols0 = sub_v[s, pl.ds(g * 16, 16)]
                for d in range(D):
                    val = plsc.load_gather(gbuf.at[buf], [rows, cols0 + d])
                    obuf[buf, d, pl.ds(g * 16, 16)] = val

            start_out(s, buf)

            @pl.when(s + 2 < S)
            def _():
                start_gather(s + 2, buf)

    wait_out(0)
    wait_out(1)


def kernel(indexes, table):
    idx_t = jnp.transpose(indexes.astype(jnp.int32))
    tbl4 = table.reshape(250000, 128)
    out = _emb(idx_t, tbl4)
    return jnp.transpose(out, (2, 0, 1))
